# baseline (speedup 1.0000x reference)
import jax
import jax.numpy as jnp
from jax import lax
from jax.experimental import pallas as pl


def _policy_kernel(state_ref, W_embed_ref, b_embed_ref, W1_ref, b1_ref,
                   W2_ref, b2_row_ref, W_dec_ref, res_ref, out_ref):
    f32 = jnp.float32
    f = W_embed_ref.shape[0]
    A = state_ref[:, f:]
    S = A.shape[0]
    Wn = A.shape[1]
    inv_S = 1.0 / S
    inv_W = 1.0 / Wn

    x_s = lax.dot_general(state_ref[:, :f], W_embed_ref[...],
                          (((1,), (0,)), ((), ())), preferred_element_type=f32)
    x_s = x_s + b_embed_ref[...]
    colsum_xs = jnp.sum(x_s, axis=0, keepdims=True)

    x_aug = jnp.concatenate([x_s, jnp.ones((S, 1), f32)], axis=1)
    P1aug = lax.dot_general(A, x_aug, (((0,), (0,)), ((), ())),
                            preferred_element_type=f32)
    P1 = P1aug[:, :x_s.shape[1]]
    c_col = P1aug[:, x_s.shape[1]:]
    agg_w1 = (P1 + colsum_xs) * inv_S
    h_w1 = jnp.maximum(
        lax.dot_general(agg_w1, W1_ref[...], (((1,), (0,)), ((), ())),
                        preferred_element_type=f32) + b1_ref[...], 0.0)

    r1 = jnp.maximum(b1_ref[...], 0.0)

    t_row = lax.dot_general(r1, W2_ref[...], (((1,), (0,)), ((), ())),
                            preferred_element_type=f32)
    cscale = 1.0 + c_col * inv_S
    h_w2 = jnp.maximum(cscale * t_row + b2_row_ref[...], 0.0)

    rowsum = jnp.sum(A, axis=1, keepdims=True)
    iota_col = lax.broadcasted_iota(jnp.int32, (S, 1), 0)
    masked = jnp.where(rowsum == 0.0, iota_col, S)
    si = jnp.min(masked)
    si = jnp.where(si >= S, 0, si)

    colsum_hw1 = jnp.sum(h_w1, axis=0, keepdims=True)
    a_row = state_ref[pl.ds(si, 1), :][:, f:]
    u1 = lax.dot_general(a_row, h_w1, (((1,), (0,)), ((), ())),
                         preferred_element_type=f32)
    agg_si = (u1 + colsum_hw1) * inv_W
    shift_h = jnp.maximum(
        lax.dot_general(agg_si, W2_ref[...], (((1,), (0,)), ((), ())),
                        preferred_element_type=f32) + b2_row_ref[...], 0.0)

    v_col = lax.dot_general(W_dec_ref[...], shift_h, (((1,), (1,)), ((), ())),
                            preferred_element_type=f32)
    scores = lax.dot_general(h_w2, v_col, (((1,), (0,)), ((), ())),
                             preferred_element_type=f32)
    scores = scores + res_ref[0, 0]

    m = jnp.max(scores, axis=0, keepdims=True)
    e = jnp.exp(scores - m)
    out_ref[...] = e / jnp.sum(e, axis=0, keepdims=True)


def kernel(state, W_embed, b_embed, W1, b1, W2, b2, W_dec, count_shifts,
           shift_features):
    f = W_embed.shape[0]
    S = state.shape[0]
    Wn = state.shape[1] - f
    D = W_embed.shape[1]
    res = ((jnp.asarray(count_shifts) - S) + (jnp.asarray(shift_features) - f))
    res = res.astype(state.dtype).reshape(1, 1)
    out = pl.pallas_call(
        _policy_kernel,
        out_shape=jax.ShapeDtypeStruct((Wn, 1), state.dtype),
    )(state, W_embed, b_embed.reshape(1, D), W1, b1.reshape(1, D),
      W2, b2.reshape(1, D), W_dec, res)
    return out.reshape(Wn)


# speedup vs baseline: 1.2743x; 1.2743x over previous
import jax
import jax.numpy as jnp
from jax import lax
from jax.experimental import pallas as pl


def _policy_kernel(state_ref, W_embed_ref, b_embed_ref, W1_ref, b1_ref,
                   W2_ref, b2_row_ref, b2_col_ref, W_dec_ref, res_ref, out_ref):
    f32 = jnp.float32
    f = W_embed_ref.shape[0]
    st = state_ref[...]
    S = st.shape[0]
    Wn = st.shape[1] - f
    inv_S = 1.0 / S
    inv_W = 1.0 / Wn

    x_s = lax.dot_general(state_ref[:, :f], W_embed_ref[...],
                          (((1,), (0,)), ((), ())), preferred_element_type=f32)
    x_s = x_s + b_embed_ref[...]
    colsum_xs = jnp.sum(x_s, axis=0, keepdims=True)

    P1f = lax.dot_general(st, x_s, (((0,), (0,)), ((), ())),
                          preferred_element_type=f32)
    P1 = P1f[f:, :]
    agg_w1 = (P1 + colsum_xs) * inv_S
    h_w1 = jnp.maximum(
        lax.dot_general(agg_w1, W1_ref[...], (((1,), (0,)), ((), ())),
                        preferred_element_type=f32) + b1_ref[...], 0.0)

    r1 = jnp.maximum(b1_ref[...], 0.0)

    c_row = 1.0 + jnp.sum(st, axis=0, keepdims=True)[:, f:] * inv_S
    t_col = lax.dot_general(W2_ref[...], r1, (((0,), (1,)), ((), ())),
                            preferred_element_type=f32)
    h_w2_T = jnp.maximum(t_col * c_row + b2_col_ref[...], 0.0)

    rowsum = (jnp.sum(st, axis=1, keepdims=True)
              - jnp.sum(state_ref[:, :f], axis=1, keepdims=True))
    iota_col = lax.broadcasted_iota(jnp.int32, (S, 1), 0)
    masked = jnp.where(rowsum == 0.0, iota_col, S)
    si = jnp.min(masked)
    si = jnp.where(si >= S, 0, si)

    colsum_hw1 = jnp.sum(h_w1, axis=0, keepdims=True)
    a_row = state_ref[pl.ds(si, 1), :][:, f:]
    u1 = lax.dot_general(a_row, h_w1, (((1,), (0,)), ((), ())),
                         preferred_element_type=f32)
    agg_si = (u1 + colsum_hw1) * inv_W
    shift_h = jnp.maximum(
        lax.dot_general(agg_si, W2_ref[...], (((1,), (0,)), ((), ())),
                        preferred_element_type=f32) + b2_row_ref[...], 0.0)

    v = lax.dot_general(shift_h, W_dec_ref[...], (((1,), (1,)), ((), ())),
                        preferred_element_type=f32)
    scores = lax.dot_general(v, h_w2_T, (((1,), (0,)), ((), ())),
                             preferred_element_type=f32)
    scores = scores + res_ref[0, 0]

    m = jnp.max(scores, axis=1, keepdims=True)
    e = jnp.exp(scores - m)
    out_ref[...] = e / jnp.sum(e, axis=1, keepdims=True)


def kernel(state, W_embed, b_embed, W1, b1, W2, b2, W_dec, count_shifts,
           shift_features):
    f = W_embed.shape[0]
    S = state.shape[0]
    Wn = state.shape[1] - f
    D = W_embed.shape[1]
    res = ((jnp.asarray(count_shifts) - S) + (jnp.asarray(shift_features) - f))
    res = res.astype(state.dtype).reshape(1, 1)
    out = pl.pallas_call(
        _policy_kernel,
        out_shape=jax.ShapeDtypeStruct((1, Wn), state.dtype),
    )(state, W_embed, b_embed.reshape(1, D), W1, b1.reshape(1, D),
      W2, b2.reshape(1, D), b2.reshape(D, 1), W_dec, res)
    return out.reshape(Wn)


# speedup vs baseline: 1.5199x; 1.1928x over previous
import jax
import jax.numpy as jnp
from jax import lax
from jax.experimental import pallas as pl


def _policy_kernel(state_ref, W_embed_ref, b_embed_ref, W1_ref, b1_ref,
                   W2_ref, b2_row_ref, b2_col_ref, W_dec_ref, out_ref):
    f32 = jnp.float32
    f = W_embed_ref.shape[0]
    st = state_ref[...]
    S = st.shape[0]
    Wn = st.shape[1] - f
    inv_S = 1.0 / S
    inv_W = 1.0 / Wn

    x_s = lax.dot_general(state_ref[:, :f], W_embed_ref[...],
                          (((1,), (0,)), ((), ())), preferred_element_type=f32)
    x_s = x_s + b_embed_ref[...]
    colsum_xs = jnp.sum(x_s, axis=0, keepdims=True)

    P1f = lax.dot_general(st, x_s, (((0,), (0,)), ((), ())),
                          preferred_element_type=f32)
    P1 = P1f[f:, :]
    agg_w1 = (P1 + colsum_xs) * inv_S
    h_w1 = jnp.maximum(
        lax.dot_general(agg_w1, W1_ref[...], (((1,), (0,)), ((), ())),
                        preferred_element_type=f32) + b1_ref[...], 0.0)

    r1 = jnp.maximum(b1_ref[...], 0.0)

    c_row = 1.0 + jnp.sum(st, axis=0, keepdims=True)[:, f:] * inv_S
    t_col = lax.dot_general(W2_ref[...], r1, (((0,), (1,)), ((), ())),
                            preferred_element_type=f32)
    h_w2_T = jnp.maximum(t_col * c_row + b2_col_ref[...], 0.0)

    rowsum = (jnp.sum(st, axis=1, keepdims=True)
              - jnp.sum(state_ref[:, :f], axis=1, keepdims=True))
    iota_col = lax.broadcasted_iota(jnp.int32, (S, 1), 0)
    masked = jnp.where(rowsum == 0.0, iota_col, S)
    si = jnp.min(masked)
    si = jnp.where(si >= S, 0, si)

    colsum_hw1 = jnp.sum(h_w1, axis=0, keepdims=True)
    a_row = state_ref[pl.ds(si, 1), :][:, f:]
    u1 = lax.dot_general(a_row, h_w1, (((1,), (0,)), ((), ())),
                         preferred_element_type=f32)
    agg_si = (u1 + colsum_hw1) * inv_W
    shift_h = jnp.maximum(
        lax.dot_general(agg_si, W2_ref[...], (((1,), (0,)), ((), ())),
                        preferred_element_type=f32) + b2_row_ref[...], 0.0)

    v = lax.dot_general(shift_h, W_dec_ref[...], (((1,), (1,)), ((), ())),
                        preferred_element_type=f32)
    scores = lax.dot_general(v, h_w2_T, (((1,), (0,)), ((), ())),
                             preferred_element_type=f32)

    m = jnp.max(scores, axis=1, keepdims=True)
    e = jnp.exp(scores - m)
    out_ref[...] = e / jnp.sum(e, axis=1, keepdims=True)


def kernel(state, W_embed, b_embed, W1, b1, W2, b2, W_dec, count_shifts,
           shift_features):
    f = W_embed.shape[0]
    S = state.shape[0]
    Wn = state.shape[1] - f
    D = W_embed.shape[1]
    out = pl.pallas_call(
        _policy_kernel,
        out_shape=jax.ShapeDtypeStruct((1, Wn), state.dtype),
    )(state, W_embed, b_embed.reshape(1, D), W1, b1.reshape(1, D),
      W2, b2.reshape(1, D), b2.reshape(D, 1), W_dec)
    return out.reshape(Wn)
